# baseline (speedup 1.0000x reference)
import numpy as np

import jax
import jax.numpy as jnp
from jax import lax
from jax.experimental import pallas as pl
from jax.experimental.pallas import tpu as pltpu

_BM = 1024

_DN_TAB = (((0,), (1,)), ((), ()))
_DN_TA = (((0,), (0,)), ((), ()))


def _body(x_ref, W1_ref, b1_ref, W2_ref, b2_ref, W3_ref, b3_ref,
          E1_ref, E2_ref, E3_ref, E4_ref, Wc1a_ref, WscT_ref, bc1e_ref,
          Wc2_ref, bc2_ref, wc3_ref, bc3_ref, out_ref):
    x = x_ref[...]
    T = x.shape[1]

    q1 = x.astype(jnp.bfloat16)
    q2 = q1 * q1
    q3 = q2 * q1
    q4 = q2 * q2
    S = (lax.dot_general(E1_ref[...], q1, _DN_TAB,
                         preferred_element_type=jnp.float32)
         + lax.dot_general(E2_ref[...], q2, _DN_TAB,
                           preferred_element_type=jnp.float32)
         + lax.dot_general(E3_ref[...], q3, _DN_TAB,
                           preferred_element_type=jnp.float32)
         + lax.dot_general(E4_ref[...], q4, _DN_TAB,
                           preferred_element_type=jnp.float32))

    s1 = S[0:1, :]
    s2 = S[1:2, :]
    s3 = S[2:3, :]
    s4 = S[3:4, :]
    mean = s1 * (1.0 / T)
    var_u = (s2 - mean * s1) * (1.0 / (T - 1))
    std = jnp.sqrt(var_u)
    m3 = (s3 - 3.0 * mean * s2 + 2.0 * mean * mean * s1) * (1.0 / T)
    m4 = (s4 - 4.0 * mean * s3 + 6.0 * mean * mean * s2
          - 3.0 * mean * mean * mean * s1) * (1.0 / T)
    std3 = std * var_u
    skew = m3 / (std3 + 1e-8)
    kurt = m4 / (var_u * var_u + 1e-8)

    mn = jnp.min(x, axis=1, keepdims=True)
    mx = jnp.max(x, axis=1, keepdims=True)
    mnmx_t = jnp.concatenate([mn, mx], axis=1).T

    stat6 = jnp.concatenate(
        [mean, std, mnmx_t, skew, kurt], axis=0).astype(jnp.bfloat16)

    h = jnp.dot(q1, W1_ref[...], preferred_element_type=jnp.float32)
    h = jax.nn.relu(h + b1_ref[...]).astype(jnp.bfloat16)
    h = jnp.dot(h, W2_ref[...], preferred_element_type=jnp.float32)
    h = jax.nn.relu(h + b2_ref[...]).astype(jnp.bfloat16)
    h = jnp.dot(h, W3_ref[...], preferred_element_type=jnp.float32)
    seq = jax.nn.relu(h + b3_ref[...]).astype(jnp.bfloat16)

    c = (lax.dot_general(Wc1a_ref[...], seq, _DN_TAB,
                         preferred_element_type=jnp.float32)
         + jnp.dot(WscT_ref[...], stat6,
                   preferred_element_type=jnp.float32))
    c = jax.nn.relu(c + bc1e_ref[...]).astype(jnp.bfloat16)
    c = lax.dot_general(Wc2_ref[...], c, _DN_TA,
                        preferred_element_type=jnp.float32)
    c = jax.nn.relu(c + bc2_ref[...]).astype(jnp.bfloat16)
    z = lax.dot_general(wc3_ref[...], c, _DN_TA,
                        preferred_element_type=jnp.float32)
    out_ref[0] = jax.nn.sigmoid(z + bc3_ref[...]) * 4.0 + 6.0


@jax.jit
def kernel(x, W1, b1, W2, b2, W3, b3, Ws, bs, Wc1, bc1, Wc2, bc2, Wc3, bc3):
    B, T = x.shape
    nb = B // _BM

    W1b = W1.astype(jnp.bfloat16)
    W2b = W2.astype(jnp.bfloat16)
    W3b = W3.astype(jnp.bfloat16)
    b1r = b1.reshape(1, -1)
    b2r = b2.reshape(1, -1)
    b3r = b3.reshape(1, -1)

    eye = np.zeros((T, 8), np.float32)
    Es = []
    for j in range(4):
        e = eye.copy()
        e[:, j] = 1.0
        Es.append(jnp.asarray(e, dtype=jnp.bfloat16))
    E1, E2, E3, E4 = Es

    Wc1b = Wc1[128:]
    Wsc_T = (Ws @ Wc1b).T.astype(jnp.bfloat16)
    bc1e = (bc1 + bs @ Wc1b).reshape(-1, 1)
    Wc1ab = Wc1[:128].astype(jnp.bfloat16)
    Wc2b = Wc2.astype(jnp.bfloat16)
    wc3b = Wc3.astype(jnp.bfloat16)
    bc2c = bc2.reshape(-1, 1)
    bc3c = bc3.reshape(1, 1)

    full = lambda a: pl.BlockSpec(a.shape, lambda i: (0,) * a.ndim)
    out = pl.pallas_call(
        _body,
        grid=(nb,),
        in_specs=[
            pl.BlockSpec((_BM, T), lambda i: (i, 0)),
            full(W1b), full(b1r), full(W2b), full(b2r), full(W3b), full(b3r),
            full(E1), full(E2), full(E3), full(E4),
            full(Wc1ab), full(Wsc_T), full(bc1e),
            full(Wc2b), full(bc2c), full(wc3b), full(bc3c),
        ],
        out_specs=pl.BlockSpec((1, 1, _BM), lambda i: (i, 0, 0)),
        out_shape=jax.ShapeDtypeStruct((nb, 1, _BM), jnp.float32),
        compiler_params=pltpu.CompilerParams(
            dimension_semantics=("parallel",),
        ),
    )(x, W1b, b1r, W2b, b2r, W3b, b3r, E1, E2, E3, E4,
      Wc1ab, Wsc_T, bc1e, Wc2b, bc2c, wc3b, bc3c)
    return out.reshape(B)


# speedup vs baseline: 1.0361x; 1.0361x over previous
import numpy as np

import jax
import jax.numpy as jnp
from jax import lax
from jax.experimental import pallas as pl
from jax.experimental.pallas import tpu as pltpu

_BM = 1024

_DN_TAB = (((0,), (1,)), ((), ()))
_DN_TA = (((0,), (0,)), ((), ()))


def _body(x_ref, W1_ref, b1_ref, W2_ref, b2_ref, W3_ref, b3_ref,
          E1_ref, E2_ref, E3_ref, E4_ref, Wc1a_ref, WscT_ref, bc1e_ref,
          Wc2_ref, bc2_ref, wc3_ref, bc3_ref, out_ref):
    x = x_ref[...]
    T = x.shape[1]

    q1 = x.astype(jnp.bfloat16)
    q2 = q1 * q1
    q3 = q2 * q1
    q4 = q2 * q2
    Sn = (jnp.dot(q1, E1_ref[...], preferred_element_type=jnp.float32)
          + jnp.dot(q2, E2_ref[...], preferred_element_type=jnp.float32)
          + jnp.dot(q3, E3_ref[...], preferred_element_type=jnp.float32)
          + jnp.dot(q4, E4_ref[...], preferred_element_type=jnp.float32))
    S = Sn.T

    s1 = S[0:1, :]
    s2 = S[1:2, :]
    s3 = S[2:3, :]
    s4 = S[3:4, :]
    mean = s1 * (1.0 / T)
    var_u = (s2 - mean * s1) * (1.0 / (T - 1))
    std = jnp.sqrt(var_u)
    m3 = (s3 - 3.0 * mean * s2 + 2.0 * mean * mean * s1) * (1.0 / T)
    m4 = (s4 - 4.0 * mean * s3 + 6.0 * mean * mean * s2
          - 3.0 * mean * mean * mean * s1) * (1.0 / T)
    std3 = std * var_u
    skew = m3 / (std3 + 1e-8)
    kurt = m4 / (var_u * var_u + 1e-8)

    mn = jnp.min(x, axis=1, keepdims=True)
    mx = jnp.max(x, axis=1, keepdims=True)
    mnmx_t = jnp.concatenate([mn, mx], axis=1).T

    stat6 = jnp.concatenate(
        [mean, std, mnmx_t, skew, kurt], axis=0).astype(jnp.bfloat16)

    h = jnp.dot(q1, W1_ref[...], preferred_element_type=jnp.float32)
    h = jax.nn.relu(h + b1_ref[...]).astype(jnp.bfloat16)
    h = jnp.dot(h, W2_ref[...], preferred_element_type=jnp.float32)
    h = jax.nn.relu(h + b2_ref[...]).astype(jnp.bfloat16)
    h = jnp.dot(h, W3_ref[...], preferred_element_type=jnp.float32)
    seq_t = jax.nn.relu(h + b3_ref[...]).T.astype(jnp.bfloat16)

    c = (lax.dot_general(Wc1a_ref[...], seq_t, _DN_TA,
                         preferred_element_type=jnp.float32)
         + jnp.dot(WscT_ref[...], stat6,
                   preferred_element_type=jnp.float32))
    c = jax.nn.relu(c + bc1e_ref[...]).astype(jnp.bfloat16)
    c = lax.dot_general(Wc2_ref[...], c, _DN_TA,
                        preferred_element_type=jnp.float32)
    c = jax.nn.relu(c + bc2_ref[...]).astype(jnp.bfloat16)
    z = lax.dot_general(wc3_ref[...], c, _DN_TA,
                        preferred_element_type=jnp.float32)
    out_ref[0] = jax.nn.sigmoid(z + bc3_ref[...]) * 4.0 + 6.0


@jax.jit
def kernel(x, W1, b1, W2, b2, W3, b3, Ws, bs, Wc1, bc1, Wc2, bc2, Wc3, bc3):
    B, T = x.shape
    nb = B // _BM

    W1b = W1.astype(jnp.bfloat16)
    W2b = W2.astype(jnp.bfloat16)
    W3b = W3.astype(jnp.bfloat16)
    b1r = b1.reshape(1, -1)
    b2r = b2.reshape(1, -1)
    b3r = b3.reshape(1, -1)

    eye = np.zeros((T, 8), np.float32)
    Es = []
    for j in range(4):
        e = eye.copy()
        e[:, j] = 1.0
        Es.append(jnp.asarray(e, dtype=jnp.bfloat16))
    E1, E2, E3, E4 = Es

    Wc1b = Wc1[128:]
    Wsc_T = (Ws @ Wc1b).T.astype(jnp.bfloat16)
    bc1e = (bc1 + bs @ Wc1b).reshape(-1, 1)
    Wc1ab = Wc1[:128].astype(jnp.bfloat16)
    Wc2b = Wc2.astype(jnp.bfloat16)
    wc3b = Wc3.astype(jnp.bfloat16)
    bc2c = bc2.reshape(-1, 1)
    bc3c = bc3.reshape(1, 1)

    full = lambda a: pl.BlockSpec(a.shape, lambda i: (0,) * a.ndim)
    out = pl.pallas_call(
        _body,
        grid=(nb,),
        in_specs=[
            pl.BlockSpec((_BM, T), lambda i: (i, 0)),
            full(W1b), full(b1r), full(W2b), full(b2r), full(W3b), full(b3r),
            full(E1), full(E2), full(E3), full(E4),
            full(Wc1ab), full(Wsc_T), full(bc1e),
            full(Wc2b), full(bc2c), full(wc3b), full(bc3c),
        ],
        out_specs=pl.BlockSpec((1, 1, _BM), lambda i: (i, 0, 0)),
        out_shape=jax.ShapeDtypeStruct((nb, 1, _BM), jnp.float32),
        compiler_params=pltpu.CompilerParams(
            dimension_semantics=("parallel",),
        ),
    )(x, W1b, b1r, W2b, b2r, W3b, b3r, E1, E2, E3, E4,
      Wc1ab, Wsc_T, bc1e, Wc2b, bc2c, wc3b, bc3c)
    return out.reshape(B)


# speedup vs baseline: 1.8204x; 1.7569x over previous
import numpy as np

import jax
import jax.numpy as jnp
from jax.experimental import pallas as pl
from jax.experimental.pallas import tpu as pltpu

_BN = 1024


def _body(xt_ref, W1s_ref, b1_ref, W2_ref, b2_ref, W3_ref, b3_ref,
          ones_ref, Wc1aT_ref, WscT_ref, bc1e_ref, Wc2T_ref, bc2_ref,
          wc3T_ref, bc3_ref, out_ref):
    xt = xt_ref[...]
    T = xt.shape[0]

    q1 = xt.astype(jnp.bfloat16)
    q2 = q1 * q1
    q3 = q2 * q1
    q4 = q2 * q2

    hs = jnp.dot(W1s_ref[...], q1, preferred_element_type=jnp.float32)
    h = jax.nn.relu(hs[0:512, :] + b1_ref[...]).astype(jnp.bfloat16)
    s1 = hs[512:513, :]

    ones_row = ones_ref[...]
    s2 = jnp.dot(ones_row, q2, preferred_element_type=jnp.float32)
    s3 = jnp.dot(ones_row, q3, preferred_element_type=jnp.float32)
    s4 = jnp.dot(ones_row, q4, preferred_element_type=jnp.float32)

    mean = s1 * (1.0 / T)
    var_u = (s2 - mean * s1) * (1.0 / (T - 1))
    std = jnp.sqrt(var_u)
    m3 = (s3 - 3.0 * mean * s2 + 2.0 * mean * mean * s1) * (1.0 / T)
    m4 = (s4 - 4.0 * mean * s3 + 6.0 * mean * mean * s2
          - 3.0 * mean * mean * mean * s1) * (1.0 / T)
    skew = m3 / (std * var_u + 1e-8)
    kurt = m4 / (var_u * var_u + 1e-8)
    mn = jnp.min(xt, axis=0, keepdims=True)
    mx = jnp.max(xt, axis=0, keepdims=True)

    stat6 = jnp.concatenate(
        [mean, std, mn, mx, skew, kurt], axis=0).astype(jnp.bfloat16)

    h = jnp.dot(W2_ref[...], h, preferred_element_type=jnp.float32)
    h = jax.nn.relu(h + b2_ref[...]).astype(jnp.bfloat16)
    h = jnp.dot(W3_ref[...], h, preferred_element_type=jnp.float32)
    seq = jax.nn.relu(h + b3_ref[...]).astype(jnp.bfloat16)

    c = (jnp.dot(Wc1aT_ref[...], seq, preferred_element_type=jnp.float32)
         + jnp.dot(WscT_ref[...], stat6,
                   preferred_element_type=jnp.float32))
    c = jax.nn.relu(c + bc1e_ref[...]).astype(jnp.bfloat16)
    c = jnp.dot(Wc2T_ref[...], c, preferred_element_type=jnp.float32)
    c = jax.nn.relu(c + bc2_ref[...]).astype(jnp.bfloat16)
    z = jnp.dot(wc3T_ref[...], c, preferred_element_type=jnp.float32)
    out_ref[0] = jax.nn.sigmoid(z + bc3_ref[...]) * 4.0 + 6.0


@jax.jit
def kernel(x, W1, b1, W2, b2, W3, b3, Ws, bs, Wc1, bc1, Wc2, bc2, Wc3, bc3):
    B, T = x.shape
    nb = B // _BN
    xt = x.T

    W1sT = jnp.concatenate(
        [W1.T, jnp.ones((1, T), jnp.float32),
         jnp.zeros((7, T), jnp.float32)], axis=0).astype(jnp.bfloat16)
    b1c = b1.reshape(-1, 1)
    W2T = W2.T.astype(jnp.bfloat16)
    b2c = b2.reshape(-1, 1)
    W3T = W3.T.astype(jnp.bfloat16)
    b3c = b3.reshape(-1, 1)
    ones_row = jnp.ones((1, T), jnp.bfloat16)

    Wc1b = Wc1[128:]
    WscT = (Ws @ Wc1b).T.astype(jnp.bfloat16)
    bc1e = (bc1 + bs @ Wc1b).reshape(-1, 1)
    Wc1aT = Wc1[:128].T.astype(jnp.bfloat16)
    Wc2T = Wc2.T.astype(jnp.bfloat16)
    wc3T = Wc3.T.astype(jnp.bfloat16)
    bc2c = bc2.reshape(-1, 1)
    bc3c = bc3.reshape(1, 1)

    full = lambda a: pl.BlockSpec(a.shape, lambda i: (0,) * a.ndim)
    out = pl.pallas_call(
        _body,
        grid=(nb,),
        in_specs=[
            pl.BlockSpec((T, _BN), lambda i: (0, i)),
            full(W1sT), full(b1c), full(W2T), full(b2c), full(W3T), full(b3c),
            full(ones_row), full(Wc1aT), full(WscT), full(bc1e),
            full(Wc2T), full(bc2c), full(wc3T), full(bc3c),
        ],
        out_specs=pl.BlockSpec((1, 1, _BN), lambda i: (i, 0, 0)),
        out_shape=jax.ShapeDtypeStruct((nb, 1, _BN), jnp.float32),
        compiler_params=pltpu.CompilerParams(
            dimension_semantics=("parallel",),
        ),
    )(xt, W1sT, b1c, W2T, b2c, W3T, b3c, ones_row,
      Wc1aT, WscT, bc1e, Wc2T, bc2c, wc3T, bc3c)
    return out.reshape(B)


# speedup vs baseline: 2.2726x; 1.2485x over previous
import numpy as np

import jax
import jax.numpy as jnp
from jax.experimental import pallas as pl
from jax.experimental.pallas import tpu as pltpu

_BN = 2048


def _body(xt_ref, W1s_ref, b1_ref, W2_ref, b2_ref, W3_ref, b3_ref,
          ones_ref, Wc1aT_ref, WscT_ref, bc1e_ref, Wc2T_ref, bc2_ref,
          wc3T_ref, bc3_ref, out_ref):
    xt = xt_ref[...]
    T = xt.shape[0]

    q1 = xt.astype(jnp.bfloat16)
    q2 = q1 * q1
    q3 = q2 * q1
    q4 = q2 * q2

    hs = jnp.dot(W1s_ref[...], q1, preferred_element_type=jnp.float32)
    h = jax.nn.relu(hs[0:512, :] + b1_ref[...]).astype(jnp.bfloat16)
    s1 = hs[512:513, :]

    ones_row = ones_ref[...]
    s2 = jnp.dot(ones_row, q2, preferred_element_type=jnp.float32)
    s3 = jnp.dot(ones_row, q3, preferred_element_type=jnp.float32)
    s4 = jnp.dot(ones_row, q4, preferred_element_type=jnp.float32)

    mean = s1 * (1.0 / T)
    var_u = (s2 - mean * s1) * (1.0 / (T - 1))
    std = jnp.sqrt(var_u)
    m3 = (s3 - 3.0 * mean * s2 + 2.0 * mean * mean * s1) * (1.0 / T)
    m4 = (s4 - 4.0 * mean * s3 + 6.0 * mean * mean * s2
          - 3.0 * mean * mean * mean * s1) * (1.0 / T)
    skew = m3 / (std * var_u + 1e-8)
    kurt = m4 / (var_u * var_u + 1e-8)
    mn = jnp.min(xt, axis=0, keepdims=True)
    mx = jnp.max(xt, axis=0, keepdims=True)

    stat6 = jnp.concatenate(
        [mean, std, mn, mx, skew, kurt], axis=0).astype(jnp.bfloat16)

    h = jnp.dot(W2_ref[...], h, preferred_element_type=jnp.float32)
    h = jax.nn.relu(h + b2_ref[...]).astype(jnp.bfloat16)
    h = jnp.dot(W3_ref[...], h, preferred_element_type=jnp.float32)
    seq = jax.nn.relu(h + b3_ref[...]).astype(jnp.bfloat16)

    c = (jnp.dot(Wc1aT_ref[...], seq, preferred_element_type=jnp.float32)
         + jnp.dot(WscT_ref[...], stat6,
                   preferred_element_type=jnp.float32))
    c = jax.nn.relu(c + bc1e_ref[...]).astype(jnp.bfloat16)
    c = jnp.dot(Wc2T_ref[...], c, preferred_element_type=jnp.float32)
    c = jax.nn.relu(c + bc2_ref[...]).astype(jnp.bfloat16)
    z = jnp.dot(wc3T_ref[...], c, preferred_element_type=jnp.float32)
    out_ref[0] = jax.nn.sigmoid(z + bc3_ref[...]) * 4.0 + 6.0


@jax.jit
def kernel(x, W1, b1, W2, b2, W3, b3, Ws, bs, Wc1, bc1, Wc2, bc2, Wc3, bc3):
    B, T = x.shape
    nb = B // _BN
    xt = x.T

    W1sT = jnp.concatenate(
        [W1.T, jnp.ones((1, T), jnp.float32),
         jnp.zeros((7, T), jnp.float32)], axis=0).astype(jnp.bfloat16)
    b1c = b1.reshape(-1, 1)
    W2T = W2.T.astype(jnp.bfloat16)
    b2c = b2.reshape(-1, 1)
    W3T = W3.T.astype(jnp.bfloat16)
    b3c = b3.reshape(-1, 1)
    ones_row = jnp.ones((1, T), jnp.bfloat16)

    Wc1b = Wc1[128:]
    WscT = (Ws @ Wc1b).T.astype(jnp.bfloat16)
    bc1e = (bc1 + bs @ Wc1b).reshape(-1, 1)
    Wc1aT = Wc1[:128].T.astype(jnp.bfloat16)
    Wc2T = Wc2.T.astype(jnp.bfloat16)
    wc3T = Wc3.T.astype(jnp.bfloat16)
    bc2c = bc2.reshape(-1, 1)
    bc3c = bc3.reshape(1, 1)

    full = lambda a: pl.BlockSpec(a.shape, lambda i: (0,) * a.ndim)
    out = pl.pallas_call(
        _body,
        grid=(nb,),
        in_specs=[
            pl.BlockSpec((T, _BN), lambda i: (0, i)),
            full(W1sT), full(b1c), full(W2T), full(b2c), full(W3T), full(b3c),
            full(ones_row), full(Wc1aT), full(WscT), full(bc1e),
            full(Wc2T), full(bc2c), full(wc3T), full(bc3c),
        ],
        out_specs=pl.BlockSpec((1, 1, _BN), lambda i: (i, 0, 0)),
        out_shape=jax.ShapeDtypeStruct((nb, 1, _BN), jnp.float32),
        compiler_params=pltpu.CompilerParams(
            dimension_semantics=("parallel",),
        ),
    )(xt, W1sT, b1c, W2T, b2c, W3T, b3c, ones_row,
      Wc1aT, WscT, bc1e, Wc2T, bc2c, wc3T, bc3c)
    return out.reshape(B)


# speedup vs baseline: 2.4600x; 1.0824x over previous
import numpy as np

import jax
import jax.numpy as jnp
from jax.experimental import pallas as pl
from jax.experimental.pallas import tpu as pltpu

_BN = 4096


def _body(xt_ref, W1s_ref, b1_ref, W2_ref, b2_ref, W3_ref, b3_ref,
          ones_ref, Wc1aT_ref, WscT_ref, bc1e_ref, Wc2T_ref, bc2_ref,
          wc3T_ref, bc3_ref, out_ref):
    xt = xt_ref[...]
    T = xt.shape[0]

    q1 = xt.astype(jnp.bfloat16)
    q2 = q1 * q1
    q3 = q2 * q1
    q4 = q2 * q2

    hs = jnp.dot(W1s_ref[...], q1, preferred_element_type=jnp.float32)
    h = jax.nn.relu(hs[0:512, :] + b1_ref[...]).astype(jnp.bfloat16)
    s1 = hs[512:513, :]

    ones_row = ones_ref[...]
    s2 = jnp.dot(ones_row, q2, preferred_element_type=jnp.float32)
    s3 = jnp.dot(ones_row, q3, preferred_element_type=jnp.float32)
    s4 = jnp.dot(ones_row, q4, preferred_element_type=jnp.float32)

    mean = s1 * (1.0 / T)
    var_u = (s2 - mean * s1) * (1.0 / (T - 1))
    std = jnp.sqrt(var_u)
    m3 = (s3 - 3.0 * mean * s2 + 2.0 * mean * mean * s1) * (1.0 / T)
    m4 = (s4 - 4.0 * mean * s3 + 6.0 * mean * mean * s2
          - 3.0 * mean * mean * mean * s1) * (1.0 / T)
    skew = m3 / (std * var_u + 1e-8)
    kurt = m4 / (var_u * var_u + 1e-8)
    mn = jnp.min(xt, axis=0, keepdims=True)
    mx = jnp.max(xt, axis=0, keepdims=True)

    stat6 = jnp.concatenate(
        [mean, std, mn, mx, skew, kurt], axis=0).astype(jnp.bfloat16)

    h = jnp.dot(W2_ref[...], h, preferred_element_type=jnp.float32)
    h = jax.nn.relu(h + b2_ref[...]).astype(jnp.bfloat16)
    h = jnp.dot(W3_ref[...], h, preferred_element_type=jnp.float32)
    seq = jax.nn.relu(h + b3_ref[...]).astype(jnp.bfloat16)

    c = (jnp.dot(Wc1aT_ref[...], seq, preferred_element_type=jnp.float32)
         + jnp.dot(WscT_ref[...], stat6,
                   preferred_element_type=jnp.float32))
    c = jax.nn.relu(c + bc1e_ref[...]).astype(jnp.bfloat16)
    c = jnp.dot(Wc2T_ref[...], c, preferred_element_type=jnp.float32)
    c = jax.nn.relu(c + bc2_ref[...]).astype(jnp.bfloat16)
    z = jnp.dot(wc3T_ref[...], c, preferred_element_type=jnp.float32)
    out_ref[0] = jax.nn.sigmoid(z + bc3_ref[...]) * 4.0 + 6.0


@jax.jit
def kernel(x, W1, b1, W2, b2, W3, b3, Ws, bs, Wc1, bc1, Wc2, bc2, Wc3, bc3):
    B, T = x.shape
    nb = B // _BN
    xt = x.T

    W1sT = jnp.concatenate(
        [W1.T, jnp.ones((1, T), jnp.float32),
         jnp.zeros((7, T), jnp.float32)], axis=0).astype(jnp.bfloat16)
    b1c = b1.reshape(-1, 1)
    W2T = W2.T.astype(jnp.bfloat16)
    b2c = b2.reshape(-1, 1)
    W3T = W3.T.astype(jnp.bfloat16)
    b3c = b3.reshape(-1, 1)
    ones_row = jnp.ones((1, T), jnp.bfloat16)

    Wc1b = Wc1[128:]
    WscT = (Ws @ Wc1b).T.astype(jnp.bfloat16)
    bc1e = (bc1 + bs @ Wc1b).reshape(-1, 1)
    Wc1aT = Wc1[:128].T.astype(jnp.bfloat16)
    Wc2T = Wc2.T.astype(jnp.bfloat16)
    wc3T = Wc3.T.astype(jnp.bfloat16)
    bc2c = bc2.reshape(-1, 1)
    bc3c = bc3.reshape(1, 1)

    full = lambda a: pl.BlockSpec(a.shape, lambda i: (0,) * a.ndim)
    out = pl.pallas_call(
        _body,
        grid=(nb,),
        in_specs=[
            pl.BlockSpec((T, _BN), lambda i: (0, i)),
            full(W1sT), full(b1c), full(W2T), full(b2c), full(W3T), full(b3c),
            full(ones_row), full(Wc1aT), full(WscT), full(bc1e),
            full(Wc2T), full(bc2c), full(wc3T), full(bc3c),
        ],
        out_specs=pl.BlockSpec((1, 1, _BN), lambda i: (i, 0, 0)),
        out_shape=jax.ShapeDtypeStruct((nb, 1, _BN), jnp.float32),
        compiler_params=pltpu.CompilerParams(
            dimension_semantics=("parallel",),
        ),
    )(xt, W1sT, b1c, W2T, b2c, W3T, b3c, ones_row,
      Wc1aT, WscT, bc1e, Wc2T, bc2c, wc3T, bc3c)
    return out.reshape(B)


# speedup vs baseline: 2.5175x; 1.0234x over previous
import numpy as np

import jax
import jax.numpy as jnp
from jax.experimental import pallas as pl
from jax.experimental.pallas import tpu as pltpu

_BN = 8192


def _body(xt_ref, W1s_ref, b1_ref, W2_ref, b2_ref, W3_ref, b3_ref,
          ones_ref, Wc1aT_ref, WscT_ref, bc1e_ref, Wc2T_ref, bc2_ref,
          wc3T_ref, bc3_ref, out_ref):
    xt = xt_ref[...]
    T = xt.shape[0]

    q1 = xt.astype(jnp.bfloat16)
    q2 = q1 * q1
    q3 = q2 * q1
    q4 = q2 * q2

    hs = jnp.dot(W1s_ref[...], q1, preferred_element_type=jnp.float32)
    h = jax.nn.relu(hs[0:512, :] + b1_ref[...]).astype(jnp.bfloat16)
    s1 = hs[512:513, :]

    ones_row = ones_ref[...]
    s2 = jnp.dot(ones_row, q2, preferred_element_type=jnp.float32)
    s3 = jnp.dot(ones_row, q3, preferred_element_type=jnp.float32)
    s4 = jnp.dot(ones_row, q4, preferred_element_type=jnp.float32)

    mean = s1 * (1.0 / T)
    var_u = (s2 - mean * s1) * (1.0 / (T - 1))
    std = jnp.sqrt(var_u)
    m3 = (s3 - 3.0 * mean * s2 + 2.0 * mean * mean * s1) * (1.0 / T)
    m4 = (s4 - 4.0 * mean * s3 + 6.0 * mean * mean * s2
          - 3.0 * mean * mean * mean * s1) * (1.0 / T)
    skew = m3 / (std * var_u + 1e-8)
    kurt = m4 / (var_u * var_u + 1e-8)
    mn = jnp.min(xt, axis=0, keepdims=True)
    mx = jnp.max(xt, axis=0, keepdims=True)

    stat6 = jnp.concatenate(
        [mean, std, mn, mx, skew, kurt], axis=0).astype(jnp.bfloat16)

    h = jnp.dot(W2_ref[...], h, preferred_element_type=jnp.float32)
    h = jax.nn.relu(h + b2_ref[...]).astype(jnp.bfloat16)
    h = jnp.dot(W3_ref[...], h, preferred_element_type=jnp.float32)
    seq = jax.nn.relu(h + b3_ref[...]).astype(jnp.bfloat16)

    c = (jnp.dot(Wc1aT_ref[...], seq, preferred_element_type=jnp.float32)
         + jnp.dot(WscT_ref[...], stat6,
                   preferred_element_type=jnp.float32))
    c = jax.nn.relu(c + bc1e_ref[...]).astype(jnp.bfloat16)
    c = jnp.dot(Wc2T_ref[...], c, preferred_element_type=jnp.float32)
    c = jax.nn.relu(c + bc2_ref[...]).astype(jnp.bfloat16)
    z = jnp.dot(wc3T_ref[...], c, preferred_element_type=jnp.float32)
    out_ref[0] = jax.nn.sigmoid(z + bc3_ref[...]) * 4.0 + 6.0


@jax.jit
def kernel(x, W1, b1, W2, b2, W3, b3, Ws, bs, Wc1, bc1, Wc2, bc2, Wc3, bc3):
    B, T = x.shape
    nb = B // _BN
    xt = x.T

    W1sT = jnp.concatenate(
        [W1.T, jnp.ones((1, T), jnp.float32),
         jnp.zeros((7, T), jnp.float32)], axis=0).astype(jnp.bfloat16)
    b1c = b1.reshape(-1, 1)
    W2T = W2.T.astype(jnp.bfloat16)
    b2c = b2.reshape(-1, 1)
    W3T = W3.T.astype(jnp.bfloat16)
    b3c = b3.reshape(-1, 1)
    ones_row = jnp.ones((1, T), jnp.bfloat16)

    Wc1b = Wc1[128:]
    WscT = (Ws @ Wc1b).T.astype(jnp.bfloat16)
    bc1e = (bc1 + bs @ Wc1b).reshape(-1, 1)
    Wc1aT = Wc1[:128].T.astype(jnp.bfloat16)
    Wc2T = Wc2.T.astype(jnp.bfloat16)
    wc3T = Wc3.T.astype(jnp.bfloat16)
    bc2c = bc2.reshape(-1, 1)
    bc3c = bc3.reshape(1, 1)

    full = lambda a: pl.BlockSpec(a.shape, lambda i: (0,) * a.ndim)
    out = pl.pallas_call(
        _body,
        grid=(nb,),
        in_specs=[
            pl.BlockSpec((T, _BN), lambda i: (0, i)),
            full(W1sT), full(b1c), full(W2T), full(b2c), full(W3T), full(b3c),
            full(ones_row), full(Wc1aT), full(WscT), full(bc1e),
            full(Wc2T), full(bc2c), full(wc3T), full(bc3c),
        ],
        out_specs=pl.BlockSpec((1, 1, _BN), lambda i: (i, 0, 0)),
        out_shape=jax.ShapeDtypeStruct((nb, 1, _BN), jnp.float32),
        compiler_params=pltpu.CompilerParams(
            dimension_semantics=("parallel",),
        ),
    )(xt, W1sT, b1c, W2T, b2c, W3T, b3c, ones_row,
      Wc1aT, WscT, bc1e, Wc2T, bc2c, wc3T, bc3c)
    return out.reshape(B)
